# baseline (speedup 1.0000x reference)
import functools

import jax
import jax.numpy as jnp
from jax import lax
from jax.experimental import pallas as pl
from jax.experimental.pallas import tpu as pltpu
from jax.experimental.pallas import tpu_sc as plsc

N, B, H = 100000, 256, 512
R = 4000
NUM_BLOCKS = N // R

NC, NS = 2, 16
NW = NC * NS
CH = 32
NROW = N // CH
RPW = -(-NROW // NW)
NROW_PAD = RPW * NW
HB = 512


def _sc_counts_body(idx3, out_n, idx_v, acc_v, all_v, red_v, sh):
    c = lax.axis_index("c")
    s = lax.axis_index("s")
    w = s * NC + c

    pltpu.sync_copy(idx3.at[w], idx_v)

    def zero(k, carry):
        acc_v[pl.ds(k * 16, 16)] = jnp.zeros((16,), jnp.float32)
        return carry

    lax.fori_loop(0, HB // 16, zero, 0)

    ones16 = jnp.ones((16,), jnp.float32)

    def body(j, carry):
        plsc.addupdate_scatter(acc_v, [idx_v[j, pl.ds(0, 16)]], ones16)
        plsc.addupdate_scatter(acc_v, [idx_v[j, pl.ds(16, 16)]], ones16)
        return carry

    lax.fori_loop(0, RPW, body, 0)

    pltpu.sync_copy(acc_v, sh.at[s])
    plsc.subcore_barrier()

    @pl.when(s == 0)
    def _reduce():
        pltpu.sync_copy(sh, all_v)

        def red(k, carry):
            t = all_v[0, pl.ds(k * 16, 16)]
            for r in range(1, NS):
                t = t + all_v[r, pl.ds(k * 16, 16)]
            red_v[pl.ds(k * 16, 16)] = t
            return carry

        lax.fori_loop(0, HB // 16, red, 0)
        pltpu.sync_copy(red_v, out_n.at[c])


_sc_counts = functools.partial(
    pl.kernel,
    out_type=jax.ShapeDtypeStruct((NC, HB), jnp.float32),
    mesh=plsc.VectorSubcoreMesh(core_axis_name="c", subcore_axis_name="s"),
    scratch_types=[
        pltpu.VMEM((RPW, CH), jnp.int32),
        pltpu.VMEM((HB,), jnp.float32),
        pltpu.VMEM((NS, HB), jnp.float32),
        pltpu.VMEM((HB,), jnp.float32),
        pltpu.VMEM_SHARED((NS, HB), jnp.float32),
    ],
    compiler_params=pltpu.CompilerParams(needs_layout_passes=False),
)


def _dense_body(ids_ref, nm_ref, nc_ref, om_ref, oc_ref):
    i = pl.program_id(0)
    ids = ids_ref[0, 0, :]
    seg = jax.lax.broadcasted_iota(jnp.int32, (B, R), 0)
    onehot = (seg == ids[None, :]).astype(jnp.float32)

    pm = jnp.dot(onehot, nm_ref[...], preferred_element_type=jnp.float32)
    pc = jnp.dot(onehot, nc_ref[...], preferred_element_type=jnp.float32)

    @pl.when(i == 0)
    def _set():
        om_ref[...] = pm
        oc_ref[...] = pc

    @pl.when(i != 0)
    def _add():
        om_ref[...] += pm
        oc_ref[...] += pc


def _finale_body(sm_ref, sc_ref, cn_ref, pm_ref, pc_ref,
                 o_tot_ref, o_m_ref, o_c_ref):
    eps = 1e-8
    cnt = cn_ref[0, 0:B] + cn_ref[1, 0:B]
    inv = 1.0 / jnp.maximum(cnt, 1.0)

    def cos_dist_mean(s, p):
        num = jnp.sum(s * p, axis=1) * inv
        na = jnp.maximum(jnp.sqrt(jnp.sum(s * s, axis=1)) * inv, eps)
        nb = jnp.maximum(jnp.sqrt(jnp.sum(p * p, axis=1)), eps)
        cos = num / (na * nb)
        return jnp.mean(1.0 - cos)

    lm = cos_dist_mean(sm_ref[...], pm_ref[...])
    lc = cos_dist_mean(sc_ref[...], pc_ref[...])
    o_m_ref[...] = jnp.reshape(lm, (1, 1))
    o_c_ref[...] = jnp.reshape(lc, (1, 1))
    o_tot_ref[...] = jnp.reshape((lm + lc) * 0.5, (1, 1))


def kernel(node_mrna, node_cnv, pooled_mrna, pooled_cnv, gene_batch):
    idx3 = jnp.pad(gene_batch, (0, NROW_PAD * CH - N),
                   constant_values=B).reshape(NW, RPW, CH)
    cnt = _sc_counts(_sc_counts_body)(idx3)

    ids3 = gene_batch.reshape(NUM_BLOCKS, 1, R)
    sums = jax.ShapeDtypeStruct((B, H), jnp.float32)
    sum_m, sum_c = pl.pallas_call(
        _dense_body,
        grid=(NUM_BLOCKS,),
        in_specs=[
            pl.BlockSpec((1, 1, R), lambda i: (i, 0, 0)),
            pl.BlockSpec((R, H), lambda i: (i, 0)),
            pl.BlockSpec((R, H), lambda i: (i, 0)),
        ],
        out_specs=[
            pl.BlockSpec((B, H), lambda i: (0, 0)),
            pl.BlockSpec((B, H), lambda i: (0, 0)),
        ],
        out_shape=[sums, sums],
    )(ids3, node_mrna, node_cnv)

    scalar = jax.ShapeDtypeStruct((1, 1), jnp.float32)
    tot, lm, lc = pl.pallas_call(
        _finale_body,
        out_shape=[scalar, scalar, scalar],
    )(sum_m, sum_c, cnt, pooled_mrna, pooled_cnv)
    return (tot[0, 0], lm[0, 0], lc[0, 0])
